# baseline (speedup 1.0000x reference)
import functools

import jax
import jax.numpy as jnp
from jax import lax
from jax.experimental import pallas as pl
from jax.experimental.pallas import tpu as pltpu
from jax.experimental.pallas import tpu_sc as plsc

VOCAB = 1000000
EMBED_DIM = 64
BATCH = 16384

_TAIL_LO = 999936
_SLAB_TC = 248
_SLAB_STRIDE_TC = 244
_SLAB_COLS = _SLAB_TC * 128
_BLK = 256
_NBLK = _SLAB_COLS // _BLK
_NBUF = 4
_PIECE = 2048
_CHUNK_ROWS = 64
_NCHUNK = 2
_STAGE_ROWS = _CHUNK_ROWS * _NCHUNK
_OUT_ROWS = BATCH + _STAGE_ROWS
_BIG = 2**30


@functools.lru_cache(maxsize=None)
def _build_gather_kernel():
    info = plsc.get_sparse_core_info()
    nc = info.num_cores
    mesh = plsc.VectorSubcoreMesh(core_axis_name="c", subcore_axis_name="s")

    @functools.partial(
        pl.kernel,
        mesh=mesh,
        out_type=jax.ShapeDtypeStruct((_OUT_ROWS, 128), jnp.float32),
        scratch_types=[
            pltpu.VMEM((_PIECE,), jnp.int32),
            pltpu.VMEM((BATCH,), jnp.int32),
            pltpu.VMEM((BATCH,), jnp.int32),
            pltpu.VMEM((_NBUF, 64, _BLK), jnp.float32),
            pltpu.VMEM((_STAGE_ROWS, 128), jnp.float32),
            pltpu.VMEM((2064,), jnp.int32),
            pltpu.VMEM((64, 64), jnp.float32),
            pltpu.VMEM((_NCHUNK, _CHUNK_ROWS), jnp.int32),
            pltpu.SemaphoreType.DMA,
            pltpu.SemaphoreType.DMA,
        ],
        compiler_params=pltpu.CompilerParams(needs_layout_passes=False),
    )
    def gather(
        idx_hbm,
        lt_hbm,
        tail_hbm,
        out2_hbm,
        piece,
        jbuf,
        cbuf,
        blockbuf,
        stage,
        lb,
        tailbuf,
        jchunk,
        sem_blk,
        sem_sc,
    ):
        iota16 = lax.iota(jnp.int32, 16)
        wid = lax.axis_index("s") * nc + lax.axis_index("c")
        c_lo = wid * (_SLAB_STRIDE_TC * 128)
        one_v = jnp.full((16,), 1, jnp.int32)

        def reinit_chunk(c):
            cv = jnp.full((16,), c, jnp.int32)
            for g in range(_CHUNK_ROWS // 16):
                icv = jnp.full((16,), g * 16, jnp.int32) + iota16
                dummy = (
                    jnp.full((16,), BATCH + g * 16, jnp.int32)
                    + cv * _CHUNK_ROWS
                    + iota16
                )
                plsc.store_scatter(jchunk, [cv, icv], dummy)

        for c in range(_NCHUNK):
            reinit_chunk(jnp.int32(c))
        for c in range(_NCHUNK):
            pltpu.async_copy(
                stage.at[pl.ds(c * _CHUNK_ROWS, _CHUNK_ROWS), :],
                out2_hbm.at[jchunk.at[jnp.int32(c)]],
                sem_sc,
            )

        def fire_blk(b, par):
            off = pl.multiple_of(c_lo + b * _BLK, 128)
            pltpu.async_copy(
                lt_hbm.at[:, pl.ds(off, _BLK)], blockbuf.at[par], sem_blk
            )

        def drain_blk():
            pltpu.make_async_copy(
                lt_hbm.at[:, pl.ds(c_lo, _BLK)], blockbuf.at[0], sem_blk
            ).wait()

        for b in range(_NBUF):
            fire_blk(jnp.int32(b), b)

        tail_lo = jnp.where(wid == 0, jnp.int32(_TAIL_LO), jnp.int32(_BIG))
        lo_v = jnp.full((16,), c_lo, jnp.int32)
        hi_v = jnp.full((16,), c_lo + _SLAB_COLS, jnp.int32)
        tail_v = jnp.full((16,), tail_lo, jnp.int32)

        def piece_body(p, cnt):
            pltpu.sync_copy(idx_hbm.at[pl.ds(p * _PIECE, _PIECE)], piece)
            base_v = jnp.full((16,), p * _PIECE, jnp.int32)

            def scan_body(g, cnt_):
                iv = piece[pl.ds(g * 16, 16)]
                jv = base_v + jnp.full((16,), g * 16, jnp.int32) + iota16
                m = ((iv >= lo_v) & (iv < hi_v)) | (iv >= tail_v)
                pm = plsc.cumsum(m.astype(jnp.int32))
                tgt = jnp.full((16,), cnt_, jnp.int32) + pm - one_v
                plsc.store_scatter(jbuf, [tgt], jv, mask=m)
                plsc.store_scatter(cbuf, [tgt], iv, mask=m)
                return cnt_ + pm[15]

            return lax.fori_loop(0, _PIECE // 16, scan_body, cnt)

        cnt = lax.fori_loop(0, BATCH // _PIECE, piece_body, jnp.int32(0))
        n_groups = (cnt + 15) // 16
        cnt_v = jnp.full((16,), cnt, jnp.int32)

        def flush(chunk):
            off = pl.multiple_of(chunk * _CHUNK_ROWS, _CHUNK_ROWS)
            pltpu.async_copy(
                stage.at[pl.ds(off, _CHUNK_ROWS), :],
                out2_hbm.at[jchunk.at[chunk]],
                sem_sc,
            )

        def drain_one_flush():
            pltpu.make_async_copy(
                stage.at[pl.ds(0, _CHUNK_ROWS), :],
                out2_hbm.at[jchunk.at[jnp.int32(0)]],
                sem_sc,
            ).wait()

        def process_block(blk_start, buf_ref, blk_w, scnt):
            blk_lo_v = jnp.full((16,), blk_start, jnp.int32)
            blk_hi_v = jnp.full((16,), blk_start + blk_w, jnp.int32)

            def cscan(g, nb):
                cols = cbuf[pl.ds(g * 16, 16)]
                pos = jnp.full((16,), g * 16, jnp.int32) + iota16
                valid = pos < cnt_v
                lm = valid & (cols >= blk_lo_v) & (cols < blk_hi_v)
                pm = plsc.cumsum(lm.astype(jnp.int32))
                tgt = jnp.full((16,), nb, jnp.int32) + pm - one_v
                plsc.store_scatter(lb, [tgt], pos, mask=lm)
                return nb + pm[15]

            nb = lax.fori_loop(0, n_groups, cscan, jnp.int32(0))

            @pl.when(nb > 0)
            def _():
                p0 = lb[pl.ds(0, 16)][0]
                lb[pl.ds(nb, 16)] = jnp.full((16,), p0, jnp.int32)

            rowq = [
                jnp.full((16,), q * 16, jnp.int32) + iota16 for q in range(4)
            ]

            def ext(g, scnt_):
                @pl.when((scnt_ & (_CHUNK_ROWS - 1)) == 0)
                def _():
                    drain_one_flush()
                    reinit_chunk((scnt_ // _CHUNK_ROWS) & (_NCHUNK - 1))

                pv = lb[pl.ds(g * 16, 16)]
                jv = plsc.load_gather(jbuf, [pv])
                colv = plsc.load_gather(cbuf, [pv]) - blk_lo_v
                slots = (
                    jnp.full((16,), scnt_, jnp.int32) + iota16
                ) & jnp.full((16,), _STAGE_ROWS - 1, jnp.int32)
                for k in range(16):
                    cbv = jnp.full((16,), colv[k], jnp.int32)
                    sbv = jnp.full((16,), slots[k], jnp.int32)
                    for q in range(4):
                        vals = plsc.load_gather(buf_ref, [rowq[q], cbv])
                        plsc.store_scatter(stage, [sbv, rowq[q]], vals)
                chunk_v = lax.shift_right_logical(
                    slots, jnp.full((16,), 6, jnp.int32)
                )
                in_chunk_v = slots & jnp.full((16,), _CHUNK_ROWS - 1, jnp.int32)
                plsc.store_scatter(jchunk, [chunk_v, in_chunk_v], jv)
                new = scnt_ + 16

                @pl.when(new & (_CHUNK_ROWS - 1) == 0)
                def _():
                    flush(((new - 1) // _CHUNK_ROWS) & (_NCHUNK - 1))

                return new

            return lax.fori_loop(0, (nb + 15) // 16, ext, scnt)

        def outer(t, scnt):
            for par in range(_NBUF):
                b = t * _NBUF + par
                drain_blk()
                scnt = process_block(
                    c_lo + b * _BLK, blockbuf.at[par], _BLK, scnt
                )
                fire_blk(jnp.minimum(b + _NBUF, _NBLK - 1), par)
            return scnt

        scnt = lax.fori_loop(0, _NBLK // _NBUF, outer, jnp.int32(0))
        for _ in range(_NBUF):
            drain_blk()

        pltpu.sync_copy(tail_hbm, tailbuf)
        scnt = process_block(jnp.int32(_TAIL_LO), tailbuf, 64, scnt)

        for c in range(_NCHUNK):
            flush(jnp.int32(c))
        for _ in range(2 * _NCHUNK - 1):
            drain_one_flush()

        @pl.when((scnt & (_CHUNK_ROWS - 1)) == 0)
        def _():
            drain_one_flush()

    return gather


def kernel(target_w_id, L):
    gather = _build_gather_kernel()
    idx = target_w_id.astype(jnp.int32)
    tail_t = lax.slice(L, (_TAIL_LO, 0), (VOCAB, EMBED_DIM)).T
    out2 = gather(idx, L.T, tail_t)
    return out2[:BATCH, :EMBED_DIM]


# speedup vs baseline: 1.1802x; 1.1802x over previous
import functools

import jax
import jax.numpy as jnp
from jax import lax
from jax.experimental import pallas as pl
from jax.experimental.pallas import tpu as pltpu
from jax.experimental.pallas import tpu_sc as plsc

VOCAB = 1000000
EMBED_DIM = 64
BATCH = 16384

_TAIL_LO = 999936
_SLAB_TC = 248
_SLAB_STRIDE_TC = 244
_SLAB_COLS = _SLAB_TC * 128
_TAIL_LOCAL = _SLAB_COLS + 512
_BLK = 256
_NBLK = _SLAB_COLS // _BLK
_NBUF = 4
_PIECE = 2048
_NRANGE = 8
_HB2_CAP = BATCH + 16 * _NRANGE
_LB_CAP = 5136
_CHUNK_ROWS = 64
_NCHUNK = 2
_STAGE_ROWS = _CHUNK_ROWS * _NCHUNK
_OUT_ROWS = BATCH + _STAGE_ROWS
_BIG = 2**30


@functools.lru_cache(maxsize=None)
def _build_gather_kernel():
    info = plsc.get_sparse_core_info()
    nc = info.num_cores
    mesh = plsc.VectorSubcoreMesh(core_axis_name="c", subcore_axis_name="s")

    @functools.partial(
        pl.kernel,
        mesh=mesh,
        out_type=jax.ShapeDtypeStruct((_OUT_ROWS, 128), jnp.float32),
        scratch_types=[
            pltpu.VMEM((_PIECE,), jnp.int32),
            pltpu.VMEM((BATCH,), jnp.int32),
            pltpu.VMEM((_HB2_CAP,), jnp.int32),
            pltpu.VMEM((16,), jnp.int32),
            pltpu.VMEM((16,), jnp.int32),
            pltpu.VMEM((_NBUF, 64, _BLK), jnp.float32),
            pltpu.VMEM((_STAGE_ROWS, 128), jnp.float32),
            pltpu.VMEM((_LB_CAP,), jnp.int32),
            pltpu.VMEM((64, 64), jnp.float32),
            pltpu.VMEM((_NCHUNK, _CHUNK_ROWS), jnp.int32),
            pltpu.SemaphoreType.DMA,
            pltpu.SemaphoreType.DMA,
        ],
        compiler_params=pltpu.CompilerParams(needs_layout_passes=False),
    )
    def gather(
        idx_hbm,
        lt_hbm,
        tail_hbm,
        out2_hbm,
        piece,
        hbuf,
        hbuf2,
        basesbuf,
        nsegbuf,
        blockbuf,
        stage,
        lb,
        tailbuf,
        jchunk,
        sem_blk,
        sem_sc,
    ):
        iota16 = lax.iota(jnp.int32, 16)
        wid = lax.axis_index("s") * nc + lax.axis_index("c")
        c_lo = wid * (_SLAB_STRIDE_TC * 128)
        one_v = jnp.full((16,), 1, jnp.int32)

        def reinit_chunk(c):
            cv = jnp.full((16,), c, jnp.int32)
            for g in range(_CHUNK_ROWS // 16):
                icv = jnp.full((16,), g * 16, jnp.int32) + iota16
                dummy = (
                    jnp.full((16,), BATCH + g * 16, jnp.int32)
                    + cv * _CHUNK_ROWS
                    + iota16
                )
                plsc.store_scatter(jchunk, [cv, icv], dummy)

        for c in range(_NCHUNK):
            reinit_chunk(jnp.int32(c))
        for c in range(_NCHUNK):
            pltpu.async_copy(
                stage.at[pl.ds(c * _CHUNK_ROWS, _CHUNK_ROWS), :],
                out2_hbm.at[jchunk.at[jnp.int32(c)]],
                sem_sc,
            )

        def fire_blk(b, par):
            off = pl.multiple_of(c_lo + b * _BLK, 128)
            pltpu.async_copy(
                lt_hbm.at[:, pl.ds(off, _BLK)], blockbuf.at[par], sem_blk
            )

        def drain_blk():
            pltpu.make_async_copy(
                lt_hbm.at[:, pl.ds(c_lo, _BLK)], blockbuf.at[0], sem_blk
            ).wait()

        for b in range(_NBUF):
            fire_blk(jnp.int32(b), b)

        tail_lo = jnp.where(wid == 0, jnp.int32(_TAIL_LO), jnp.int32(_BIG))
        lo_v = jnp.full((16,), c_lo, jnp.int32)
        hi_v = jnp.full((16,), c_lo + _SLAB_COLS, jnp.int32)
        tail_v = jnp.full((16,), tail_lo, jnp.int32)
        tail_sub_v = jnp.full((16,), _TAIL_LO - _TAIL_LOCAL, jnp.int32)
        fourteen_v = jnp.full((16,), 14, jnp.int32)

        def piece_body(p, cnt):
            pltpu.sync_copy(idx_hbm.at[pl.ds(p * _PIECE, _PIECE)], piece)
            base_v = jnp.full((16,), p * _PIECE, jnp.int32)

            def scan_body(g, cnt_):
                iv = piece[pl.ds(g * 16, 16)]
                jv = base_v + jnp.full((16,), g * 16, jnp.int32) + iota16
                is_tail = iv >= tail_v
                m = ((iv >= lo_v) & (iv < hi_v)) | is_tail
                local = iv - jnp.where(is_tail, tail_sub_v, lo_v)
                packed = lax.shift_left(local, fourteen_v) | jv
                pm = plsc.cumsum(m.astype(jnp.int32))
                tgt = jnp.full((16,), cnt_, jnp.int32) + pm - one_v
                plsc.store_scatter(hbuf, [tgt], packed, mask=m)
                return cnt_ + pm[15]

            return lax.fori_loop(0, _PIECE // 16, scan_body, cnt)

        cnt = lax.fori_loop(0, BATCH // _PIECE, piece_body, jnp.int32(0))
        n_groups = (cnt + 15) // 16
        cnt_v = jnp.full((16,), cnt, jnp.int32)

        def count_body(g, counts):
            words = hbuf[pl.ds(g * 16, 16)]
            pos = jnp.full((16,), g * 16, jnp.int32) + iota16
            valid = pos < cnt_v
            rv = lax.shift_right_logical(words, jnp.full((16,), 26, jnp.int32))
            out = []
            for r in range(_NRANGE):
                mr = valid & (rv == jnp.full((16,), r, jnp.int32))
                out.append(
                    counts[r] + plsc.all_reduce_population_count(mr)[0]
                )
            return tuple(out)

        counts = lax.fori_loop(
            0, n_groups, count_body, tuple(jnp.int32(0) for _ in range(_NRANGE))
        )
        bases = []
        acc = jnp.int32(0)
        for r in range(_NRANGE):
            bases.append(acc)
            acc = acc + ((counts[r] + 15) // 16) * 16
        for r in range(_NRANGE):
            plsc.store_scatter(
                basesbuf, [jnp.full((16,), r, jnp.int32)],
                jnp.full((16,), bases[r], jnp.int32),
            )
            plsc.store_scatter(
                nsegbuf, [jnp.full((16,), r, jnp.int32)],
                jnp.full((16,), counts[r], jnp.int32),
            )

        def sort_body(g, ctrs):
            words = hbuf[pl.ds(g * 16, 16)]
            pos = jnp.full((16,), g * 16, jnp.int32) + iota16
            valid = pos < cnt_v
            rv = lax.shift_right_logical(words, jnp.full((16,), 26, jnp.int32))
            out = []
            for r in range(_NRANGE):
                mr = valid & (rv == jnp.full((16,), r, jnp.int32))
                pm = plsc.cumsum(mr.astype(jnp.int32))
                tgt = jnp.full((16,), ctrs[r], jnp.int32) + pm - one_v
                plsc.store_scatter(hbuf2, [tgt], words, mask=mr)
                out.append(ctrs[r] + pm[15])
            return tuple(out)

        lax.fori_loop(0, n_groups, sort_body, tuple(bases))

        def flush(chunk):
            off = pl.multiple_of(chunk * _CHUNK_ROWS, _CHUNK_ROWS)
            pltpu.async_copy(
                stage.at[pl.ds(off, _CHUNK_ROWS), :],
                out2_hbm.at[jchunk.at[chunk]],
                sem_sc,
            )

        def drain_one_flush():
            pltpu.make_async_copy(
                stage.at[pl.ds(0, _CHUNK_ROWS), :],
                out2_hbm.at[jchunk.at[jnp.int32(0)]],
                sem_sc,
            ).wait()

        def process_block(blk_lo, rng, buf_ref, blk_w, scnt):
            rng_v = jnp.full((16,), rng, jnp.int32)
            seg_base = plsc.load_gather(basesbuf, [rng_v])[0]
            seg_n = plsc.load_gather(nsegbuf, [rng_v])[0]
            seg_n_v = jnp.full((16,), seg_n, jnp.int32)
            blk_lo_v = jnp.full((16,), blk_lo, jnp.int32)
            blk_hi_v = jnp.full((16,), blk_lo + blk_w, jnp.int32)
            cap_v = jnp.full((16,), _LB_CAP - 1, jnp.int32)

            def cscan(g, nb):
                words = hbuf2[pl.ds(seg_base + g * 16, 16)]
                pos = jnp.full((16,), g * 16, jnp.int32) + iota16
                valid = pos < seg_n_v
                local = lax.shift_right_logical(words, fourteen_v)
                lm = valid & (local >= blk_lo_v) & (local < blk_hi_v)
                pm = plsc.cumsum(lm.astype(jnp.int32))
                tgt = jnp.minimum(
                    jnp.full((16,), nb, jnp.int32) + pm - one_v, cap_v
                )
                plsc.store_scatter(lb, [tgt], words, mask=lm)
                return nb + pm[15]

            nb = lax.fori_loop(0, (seg_n + 15) // 16, cscan, jnp.int32(0))

            @pl.when(nb > 0)
            def _():
                p0 = lb[pl.ds(0, 16)][0]
                lb[pl.ds(nb, 16)] = jnp.full((16,), p0, jnp.int32)

            rowq = [
                jnp.full((16,), q * 16, jnp.int32) + iota16 for q in range(4)
            ]
            jmask_v = jnp.full((16,), BATCH - 1, jnp.int32)

            def ext(g, scnt_):
                @pl.when((scnt_ & (_CHUNK_ROWS - 1)) == 0)
                def _():
                    drain_one_flush()
                    reinit_chunk((scnt_ // _CHUNK_ROWS) & (_NCHUNK - 1))

                pw = lb[pl.ds(g * 16, 16)]
                jv = pw & jmask_v
                colv = lax.shift_right_logical(pw, fourteen_v) - blk_lo_v
                slots = (
                    jnp.full((16,), scnt_, jnp.int32) + iota16
                ) & jnp.full((16,), _STAGE_ROWS - 1, jnp.int32)
                for k in range(16):
                    cbv = jnp.full((16,), colv[k], jnp.int32)
                    sbv = jnp.full((16,), slots[k], jnp.int32)
                    for q in range(4):
                        vals = plsc.load_gather(buf_ref, [rowq[q], cbv])
                        plsc.store_scatter(stage, [sbv, rowq[q]], vals)
                chunk_v = lax.shift_right_logical(
                    slots, jnp.full((16,), 6, jnp.int32)
                )
                in_chunk_v = slots & jnp.full((16,), _CHUNK_ROWS - 1, jnp.int32)
                plsc.store_scatter(jchunk, [chunk_v, in_chunk_v], jv)
                new = scnt_ + 16

                @pl.when(new & (_CHUNK_ROWS - 1) == 0)
                def _():
                    flush(((new - 1) // _CHUNK_ROWS) & (_NCHUNK - 1))

                return new

            return lax.fori_loop(0, (nb + 15) // 16, ext, scnt)

        def outer(t, scnt):
            for par in range(_NBUF):
                b = t * _NBUF + par
                drain_blk()
                scnt = process_block(
                    b * _BLK, b // 16, blockbuf.at[par], _BLK, scnt
                )
                fire_blk(jnp.minimum(b + _NBUF, _NBLK - 1), par)
            return scnt

        scnt = lax.fori_loop(0, _NBLK // _NBUF, outer, jnp.int32(0))
        for _ in range(_NBUF):
            drain_blk()

        pltpu.sync_copy(tail_hbm, tailbuf)
        scnt = process_block(
            jnp.int32(_TAIL_LOCAL), jnp.int32(_NRANGE - 1), tailbuf, 64, scnt
        )

        for c in range(_NCHUNK):
            flush(jnp.int32(c))
        for _ in range(2 * _NCHUNK - 1):
            drain_one_flush()

        @pl.when((scnt & (_CHUNK_ROWS - 1)) == 0)
        def _():
            drain_one_flush()

    return gather


def kernel(target_w_id, L):
    gather = _build_gather_kernel()
    idx = target_w_id.astype(jnp.int32)
    tail_t = lax.slice(L, (_TAIL_LO, 0), (VOCAB, EMBED_DIM)).T
    out2 = gather(idx, L.T, tail_t)
    return out2[:BATCH, :EMBED_DIM]


# speedup vs baseline: 1.3635x; 1.1554x over previous
import functools

import jax
import jax.numpy as jnp
from jax import lax
from jax.experimental import pallas as pl
from jax.experimental.pallas import tpu as pltpu
from jax.experimental.pallas import tpu_sc as plsc

VOCAB = 1000000
EMBED_DIM = 64
BATCH = 16384

_TAIL_LO = 999936
_SLAB_TC = 248
_SLAB_STRIDE_TC = 244
_SLAB_COLS = _SLAB_TC * 128
_TAIL_LOCAL = _SLAB_COLS + 512
_BLK = 512
_NBLK = _SLAB_COLS // _BLK
_NBUF = 2
_BLK_PER_RNG = 4096 // _BLK
_PIECE = 2048
_NRANGE = 8
_HB2_CAP = BATCH + 16 * _NRANGE
_LB_CAP = 5136
_CHUNK_ROWS = 64
_NCHUNK = 2
_STAGE_ROWS = _CHUNK_ROWS * _NCHUNK
_OUT_ROWS = BATCH + _STAGE_ROWS
_BIG = 2**30


@functools.lru_cache(maxsize=None)
def _build_gather_kernel():
    info = plsc.get_sparse_core_info()
    nc = info.num_cores
    mesh = plsc.VectorSubcoreMesh(core_axis_name="c", subcore_axis_name="s")

    @functools.partial(
        pl.kernel,
        mesh=mesh,
        out_type=jax.ShapeDtypeStruct((_OUT_ROWS, 128), jnp.float32),
        scratch_types=[
            pltpu.VMEM((_PIECE,), jnp.int32),
            pltpu.VMEM((BATCH,), jnp.int32),
            pltpu.VMEM((_HB2_CAP,), jnp.int32),
            pltpu.VMEM((16,), jnp.int32),
            pltpu.VMEM((16,), jnp.int32),
            pltpu.VMEM((_NBUF, 64, _BLK), jnp.float32),
            pltpu.VMEM((_STAGE_ROWS, 128), jnp.float32),
            pltpu.VMEM((_LB_CAP,), jnp.int32),
            pltpu.VMEM((64, 64), jnp.float32),
            pltpu.VMEM((_NCHUNK, _CHUNK_ROWS), jnp.int32),
            pltpu.SemaphoreType.DMA,
            pltpu.SemaphoreType.DMA,
        ],
        compiler_params=pltpu.CompilerParams(needs_layout_passes=False),
    )
    def gather(
        idx_hbm,
        lt_hbm,
        tail_hbm,
        out2_hbm,
        piece,
        hbuf,
        hbuf2,
        basesbuf,
        nsegbuf,
        blockbuf,
        stage,
        lb,
        tailbuf,
        jchunk,
        sem_blk,
        sem_sc,
    ):
        iota16 = lax.iota(jnp.int32, 16)
        wid = lax.axis_index("s") * nc + lax.axis_index("c")
        c_lo = wid * (_SLAB_STRIDE_TC * 128)
        one_v = jnp.full((16,), 1, jnp.int32)

        def reinit_chunk(c):
            cv = jnp.full((16,), c, jnp.int32)
            for g in range(_CHUNK_ROWS // 16):
                icv = jnp.full((16,), g * 16, jnp.int32) + iota16
                dummy = (
                    jnp.full((16,), BATCH + g * 16, jnp.int32)
                    + cv * _CHUNK_ROWS
                    + iota16
                )
                plsc.store_scatter(jchunk, [cv, icv], dummy)

        for c in range(_NCHUNK):
            reinit_chunk(jnp.int32(c))
        for c in range(_NCHUNK):
            pltpu.async_copy(
                stage.at[pl.ds(c * _CHUNK_ROWS, _CHUNK_ROWS), :],
                out2_hbm.at[jchunk.at[jnp.int32(c)]],
                sem_sc,
            )

        def fire_blk(b, par):
            off = pl.multiple_of(c_lo + b * _BLK, 128)
            pltpu.async_copy(
                lt_hbm.at[:, pl.ds(off, _BLK)], blockbuf.at[par], sem_blk
            )

        def drain_blk():
            pltpu.make_async_copy(
                lt_hbm.at[:, pl.ds(c_lo, _BLK)], blockbuf.at[0], sem_blk
            ).wait()

        for b in range(_NBUF):
            fire_blk(jnp.int32(b), b)

        tail_lo = jnp.where(wid == 0, jnp.int32(_TAIL_LO), jnp.int32(_BIG))
        lo_v = jnp.full((16,), c_lo, jnp.int32)
        hi_v = jnp.full((16,), c_lo + _SLAB_COLS, jnp.int32)
        tail_v = jnp.full((16,), tail_lo, jnp.int32)
        tail_sub_v = jnp.full((16,), _TAIL_LO - _TAIL_LOCAL, jnp.int32)
        fourteen_v = jnp.full((16,), 14, jnp.int32)

        def piece_body(p, cnt):
            pltpu.sync_copy(idx_hbm.at[pl.ds(p * _PIECE, _PIECE)], piece)
            base_v = jnp.full((16,), p * _PIECE, jnp.int32)

            def scan_body(g, cnt_):
                iv = piece[pl.ds(g * 16, 16)]
                jv = base_v + jnp.full((16,), g * 16, jnp.int32) + iota16
                is_tail = iv >= tail_v
                m = ((iv >= lo_v) & (iv < hi_v)) | is_tail
                local = iv - jnp.where(is_tail, tail_sub_v, lo_v)
                packed = lax.shift_left(local, fourteen_v) | jv
                pm = plsc.cumsum(m.astype(jnp.int32))
                tgt = jnp.full((16,), cnt_, jnp.int32) + pm - one_v
                plsc.store_scatter(hbuf, [tgt], packed, mask=m)
                return cnt_ + pm[15]

            return lax.fori_loop(0, _PIECE // 16, scan_body, cnt)

        cnt = lax.fori_loop(0, BATCH // _PIECE, piece_body, jnp.int32(0))
        n_groups = (cnt + 15) // 16
        cnt_v = jnp.full((16,), cnt, jnp.int32)

        def count_body(g, counts):
            words = hbuf[pl.ds(g * 16, 16)]
            pos = jnp.full((16,), g * 16, jnp.int32) + iota16
            valid = pos < cnt_v
            rv = lax.shift_right_logical(words, jnp.full((16,), 26, jnp.int32))
            out = []
            for r in range(_NRANGE):
                mr = valid & (rv == jnp.full((16,), r, jnp.int32))
                out.append(
                    counts[r] + plsc.all_reduce_population_count(mr)[0]
                )
            return tuple(out)

        counts = lax.fori_loop(
            0, n_groups, count_body, tuple(jnp.int32(0) for _ in range(_NRANGE))
        )
        bases = []
        acc = jnp.int32(0)
        for r in range(_NRANGE):
            bases.append(acc)
            acc = acc + ((counts[r] + 15) // 16) * 16
        for r in range(_NRANGE):
            plsc.store_scatter(
                basesbuf, [jnp.full((16,), r, jnp.int32)],
                jnp.full((16,), bases[r], jnp.int32),
            )
            plsc.store_scatter(
                nsegbuf, [jnp.full((16,), r, jnp.int32)],
                jnp.full((16,), counts[r], jnp.int32),
            )

        def sort_body(g, ctrs):
            words = hbuf[pl.ds(g * 16, 16)]
            pos = jnp.full((16,), g * 16, jnp.int32) + iota16
            valid = pos < cnt_v
            rv = lax.shift_right_logical(words, jnp.full((16,), 26, jnp.int32))
            out = []
            for r in range(_NRANGE):
                mr = valid & (rv == jnp.full((16,), r, jnp.int32))
                pm = plsc.cumsum(mr.astype(jnp.int32))
                tgt = jnp.full((16,), ctrs[r], jnp.int32) + pm - one_v
                plsc.store_scatter(hbuf2, [tgt], words, mask=mr)
                out.append(ctrs[r] + pm[15])
            return tuple(out)

        lax.fori_loop(0, n_groups, sort_body, tuple(bases))

        def flush(chunk):
            off = pl.multiple_of(chunk * _CHUNK_ROWS, _CHUNK_ROWS)
            pltpu.async_copy(
                stage.at[pl.ds(off, _CHUNK_ROWS), :],
                out2_hbm.at[jchunk.at[chunk]],
                sem_sc,
            )

        def drain_one_flush():
            pltpu.make_async_copy(
                stage.at[pl.ds(0, _CHUNK_ROWS), :],
                out2_hbm.at[jchunk.at[jnp.int32(0)]],
                sem_sc,
            ).wait()

        def process_block(blk_lo, rng, buf_ref, blk_w, scnt):
            rng_v = jnp.full((16,), rng, jnp.int32)
            seg_base = plsc.load_gather(basesbuf, [rng_v])[0]
            seg_n = plsc.load_gather(nsegbuf, [rng_v])[0]
            seg_n_v = jnp.full((16,), seg_n, jnp.int32)
            blk_lo_v = jnp.full((16,), blk_lo, jnp.int32)
            blk_hi_v = jnp.full((16,), blk_lo + blk_w, jnp.int32)
            cap_v = jnp.full((16,), _LB_CAP - 1, jnp.int32)

            def cscan(g, nb):
                words = hbuf2[pl.ds(seg_base + g * 16, 16)]
                pos = jnp.full((16,), g * 16, jnp.int32) + iota16
                valid = pos < seg_n_v
                local = lax.shift_right_logical(words, fourteen_v)
                lm = valid & (local >= blk_lo_v) & (local < blk_hi_v)
                pm = plsc.cumsum(lm.astype(jnp.int32))
                tgt = jnp.minimum(
                    jnp.full((16,), nb, jnp.int32) + pm - one_v, cap_v
                )
                plsc.store_scatter(lb, [tgt], words, mask=lm)
                return nb + pm[15]

            nb = lax.fori_loop(0, (seg_n + 15) // 16, cscan, jnp.int32(0))

            @pl.when(nb > 0)
            def _():
                p0 = lb[pl.ds(0, 16)][0]
                lb[pl.ds(nb, 16)] = jnp.full((16,), p0, jnp.int32)

            rowq = [
                jnp.full((16,), q * 16, jnp.int32) + iota16 for q in range(4)
            ]
            jmask_v = jnp.full((16,), BATCH - 1, jnp.int32)

            def ext(g, scnt_):
                @pl.when((scnt_ & (_CHUNK_ROWS - 1)) == 0)
                def _():
                    drain_one_flush()
                    reinit_chunk((scnt_ // _CHUNK_ROWS) & (_NCHUNK - 1))

                pw = lb[pl.ds(g * 16, 16)]
                jv = pw & jmask_v
                colv = lax.shift_right_logical(pw, fourteen_v) - blk_lo_v
                slots = (
                    jnp.full((16,), scnt_, jnp.int32) + iota16
                ) & jnp.full((16,), _STAGE_ROWS - 1, jnp.int32)
                for k in range(16):
                    cbv = jnp.full((16,), colv[k], jnp.int32)
                    sbv = jnp.full((16,), slots[k], jnp.int32)
                    for q in range(4):
                        vals = plsc.load_gather(buf_ref, [rowq[q], cbv])
                        plsc.store_scatter(stage, [sbv, rowq[q]], vals)
                chunk_v = lax.shift_right_logical(
                    slots, jnp.full((16,), 6, jnp.int32)
                )
                in_chunk_v = slots & jnp.full((16,), _CHUNK_ROWS - 1, jnp.int32)
                plsc.store_scatter(jchunk, [chunk_v, in_chunk_v], jv)
                new = scnt_ + 16

                @pl.when(new & (_CHUNK_ROWS - 1) == 0)
                def _():
                    flush(((new - 1) // _CHUNK_ROWS) & (_NCHUNK - 1))

                return new

            return lax.fori_loop(0, (nb + 15) // 16, ext, scnt)

        def outer(t, scnt):
            for par in range(_NBUF):
                b = t * _NBUF + par
                drain_blk()
                scnt = process_block(
                    b * _BLK, b // _BLK_PER_RNG, blockbuf.at[par], _BLK, scnt
                )
                fire_blk(jnp.minimum(b + _NBUF, _NBLK - 1), par)
            return scnt

        scnt = lax.fori_loop(0, _NBLK // _NBUF, outer, jnp.int32(0))
        for _ in range(_NBUF):
            drain_blk()

        pltpu.sync_copy(tail_hbm, tailbuf)
        scnt = process_block(
            jnp.int32(_TAIL_LOCAL), jnp.int32(_NRANGE - 1), tailbuf, 64, scnt
        )

        for c in range(_NCHUNK):
            flush(jnp.int32(c))
        for _ in range(2 * _NCHUNK - 1):
            drain_one_flush()

        @pl.when((scnt & (_CHUNK_ROWS - 1)) == 0)
        def _():
            drain_one_flush()

    return gather


def kernel(target_w_id, L):
    gather = _build_gather_kernel()
    idx = target_w_id.astype(jnp.int32)
    tail_t = lax.slice(L, (_TAIL_LO, 0), (VOCAB, EMBED_DIM)).T
    out2 = gather(idx, L.T, tail_t)
    return out2[:BATCH, :EMBED_DIM]
